# baseline (speedup 1.0000x reference)
import functools

import numpy as np
import jax
import jax.numpy as jnp
from jax import lax
from jax.experimental import pallas as pl
from jax.experimental.pallas import tpu as pltpu
from jax.experimental.pallas import tpu_sc as plsc

N_CELLS = 2
C = 128
H = 256
W = 256
NPTS = 131072

NC = 2
NS = 16
NW = NC * NS
PTS_PER_W = NPTS // NW
CHUNK = 64
N_CHUNKS = PTS_PER_W // CHUNK
IDX_ROWS = CHUNK * 4 // 128
RPC = CHUNK * 4
NPIX = N_CELLS * H * W

_CONV_T = 8


def _conv_tanh_body(wt_ref, prev_ref, cur_ref, nxt_ref, out_ref):
    i = pl.program_id(1)
    ni = pl.num_programs(1)
    t = _CONV_T
    cur = cur_ref[0]
    top = jnp.where(i > 0, prev_ref[0], 0.0)
    bot = jnp.where(i < ni - 1, nxt_ref[0], 0.0)
    ext = jnp.concatenate([top, cur, bot], axis=0)
    acc = jnp.zeros((t, W, C), jnp.float32)
    zcol = jnp.zeros((t, 1, C), jnp.float32)
    for dh in range(3):
        sl = ext[dh:dh + t]
        for dw in range(3):
            if dw == 0:
                sh = jnp.concatenate([zcol, sl[:, :W - 1, :]], axis=1)
            elif dw == 2:
                sh = jnp.concatenate([sl[:, 1:, :], zcol], axis=1)
            else:
                sh = sl
            acc = acc + sh * wt_ref[dh * 3 + dw][None, None, :]
    u = jax.lax.bitcast_convert_type(jnp.tanh(acc), jnp.uint32)
    bf = jnp.right_shift(
        u + jnp.uint32(0x7FFF)
        + jnp.bitwise_and(jnp.right_shift(u, 16), jnp.uint32(1)), 16)
    g = bf.reshape(t, W // 2, 2, C)
    lo = g[:, :, 0, :]
    hi = g[:, :, 1, :]
    zrow = jnp.zeros((t, 1, C), jnp.uint32)
    lo_next = jnp.concatenate([lo[:, 1:, :], zrow], axis=1)
    e_word = jnp.bitwise_or(lo, jnp.left_shift(hi, 16))
    o_word = jnp.bitwise_or(hi, jnp.left_shift(lo_next, 16))
    out_ref[0, 0] = jax.lax.bitcast_convert_type(e_word, jnp.int32)
    out_ref[1, 0] = jax.lax.bitcast_convert_type(o_word, jnp.int32)


def _conv_tanh(f_nhwc, wt9):
    nblk = H // _CONV_T
    blk = (1, _CONV_T, W, C)
    return pl.pallas_call(
        _conv_tanh_body,
        grid=(N_CELLS, nblk),
        in_specs=[
            pl.BlockSpec((9, C), lambda n, i: (0, 0)),
            pl.BlockSpec((1, 1, W, C),
                         lambda n, i: (n, jnp.maximum(i * _CONV_T - 1, 0), 0, 0)),
            pl.BlockSpec(blk, lambda n, i: (n, i, 0, 0)),
            pl.BlockSpec((1, 1, W, C),
                         lambda n, i: (n, jnp.minimum(i * _CONV_T + _CONV_T,
                                                      H - 1), 0, 0)),
        ],
        out_specs=pl.BlockSpec(
            (2, 1, _CONV_T, W // 2, C), lambda n, i: (0, n, i, 0, 0)),
        out_shape=jax.ShapeDtypeStruct(
            (2, N_CELLS, H, W // 2, C), jnp.int32),
    )(wt9, f_nhwc, f_nhwc, f_nhwc)


def _idxwts_body(x_ref, y_ref, idx_ref, wts_ref):
    xv = x_ref[...]
    yv = y_ref[...]
    ix0 = xv * jnp.float32(W - 2) * 0.5
    iy0 = yv * jnp.float32(H - 2)
    for n in range(N_CELLS):
        off = jnp.float32(n / N_CELLS)
        ix = ix0 + off
        iy = iy0 + off
        ixl = jnp.floor(ix)
        iyt = jnp.floor(iy)
        fx = ix - ixl
        fy = iy - iyt
        wxr = 0.5 - 0.5 * jnp.cos(jnp.pi * fx)
        wxl = 1.0 - wxr
        wyb = 0.5 - 0.5 * jnp.cos(jnp.pi * fy)
        wyt = 1.0 - wyb
        ixl_i = jnp.clip(ixl, 0, W - 1).astype(jnp.int32)
        iyt_i = jnp.clip(iyt, 0, H - 1).astype(jnp.int32)
        p_top = n * H * W + iyt_i * W + ixl_i
        par = jnp.bitwise_and(p_top, 1)
        r_top = jnp.right_shift(p_top, 1) + par * (NPIX // 2)
        idx_ref[2 * n + 0] = r_top
        idx_ref[2 * n + 1] = r_top + W // 2
        wts_ref[4 * n + 0] = wxl * wyt
        wts_ref[4 * n + 1] = wxr * wyt
        wts_ref[4 * n + 2] = wxl * wyb
        wts_ref[4 * n + 3] = wxr * wyb


def _idxwts(xb, yb):
    rows = NPTS // 128
    rblk = 128
    nblk = rows // rblk
    return pl.pallas_call(
        _idxwts_body,
        grid=(nblk,),
        in_specs=[
            pl.BlockSpec((rblk, 128), lambda i: (i, 0)),
            pl.BlockSpec((rblk, 128), lambda i: (i, 0)),
        ],
        out_specs=[
            pl.BlockSpec((4, rblk, 128), lambda i: (0, i, 0)),
            pl.BlockSpec((8, rblk, 128), lambda i: (0, i, 0)),
        ],
        out_shape=[
            jax.ShapeDtypeStruct((4, rows, 128), jnp.int32),
            jax.ShapeDtypeStruct((8, rows, 128), jnp.float32),
        ],
    )(xb, yb)


def _sc_interp_body(table_hbm, idx_hbm, wts_hbm, out_hbm,
                    idx_v, wts_v, rows_v, out_v,
                    sem_idx, sem_gat, sem_out):
    wid = lax.axis_index("s") * NC + lax.axis_index("c")
    iota = lax.iota(jnp.int32, 16)
    n = N_CHUNKS

    def idx_copies(k, b):
        g4 = wid * (N_CHUNKS * IDX_ROWS) + k * IDX_ROWS
        return (
            pltpu.make_async_copy(
                idx_hbm.at[pl.ds(g4, IDX_ROWS)], idx_v.at[b], sem_idx),
            pltpu.make_async_copy(
                wts_hbm.at[pl.ds(wid * PTS_PER_W + k * CHUNK, CHUNK)],
                wts_v.at[pl.ds(b * CHUNK, CHUNK)], sem_idx),
        )

    def gather_copies(b):
        return [pltpu.make_async_copy(
                    table_hbm.at[idx_v.at[b, i]],
                    rows_v.at[pl.ds((b * IDX_ROWS + i) * 128, 128)],
                    sem_gat)
                for i in range(IDX_ROWS)]

    def out_copy(k, b):
        return pltpu.make_async_copy(
            out_v.at[pl.ds(b * CHUNK, CHUNK)],
            out_hbm.at[pl.ds(wid * PTS_PER_W + k * CHUNK, CHUNK)],
            sem_out)

    def compute(b):
        boff = b * CHUNK
        rbase = b * RPC

        @plsc.parallel_loop(0, CHUNK, step=1, unroll=4)
        def pbody(p):
            r0 = rbase + p * 4
            wrow = boff + p
            ws = [wts_v[wrow, pl.ds(j * 16, 16)] for j in range(8)]
            for u in range(8):
                acc = jnp.zeros((16,), jnp.float32)
                for q in range(4):
                    packed = plsc.bitcast(
                        rows_v[r0 + q, pl.ds(u * 16, 16)], jnp.bfloat16)
                    a, bb = plsc.unpack(
                        packed, format=plsc.PackFormat.INTERLEAVED)
                    acc = acc + ws[2 * q] * a + ws[2 * q + 1] * bb
                out_v[wrow, pl.ds(u * 16, 16)] = acc

    for cp in idx_copies(0, 0):
        cp.start()
    for cp in idx_copies(0, 0):
        cp.wait()
    for cp in gather_copies(0):
        cp.start()
    for cp in idx_copies(1, 1):
        cp.start()

    def loop_body(k, carry):
        b = jnp.bitwise_and(k, 1)
        nb = 1 - b
        for cp in gather_copies(b):
            cp.wait()

        @pl.when(k + 1 < n)
        def _():
            for cp in idx_copies(k + 1, nb):
                cp.wait()
            for cp in gather_copies(nb):
                cp.start()

        @pl.when(k >= 2)
        def _():
            out_copy(k - 2, b).wait()

        compute(b)

        @pl.when(k + 2 < n)
        def _():
            for cp in idx_copies(k + 2, b):
                cp.start()

        out_copy(k, b).start()
        return carry

    lax.fori_loop(0, n, loop_body, 0)
    out_copy(n - 2, (n - 2) & 1).wait()
    out_copy(n - 1, (n - 1) & 1).wait()


@functools.lru_cache(maxsize=1)
def _sc_interp():
    return pl.kernel(
        _sc_interp_body,
        out_type=jax.ShapeDtypeStruct((NPTS, C), jnp.float32),
        mesh=plsc.VectorSubcoreMesh(
            core_axis_name="c", subcore_axis_name="s",
            num_cores=NC, num_subcores=NS),
        compiler_params=pltpu.CompilerParams(needs_layout_passes=False),
        scratch_types=[
            pltpu.VMEM((2, IDX_ROWS, 128), jnp.int32),
            pltpu.VMEM((2 * CHUNK, C), jnp.float32),
            pltpu.VMEM((2 * RPC, 128), jnp.int32),
            pltpu.VMEM((2 * CHUNK, C), jnp.float32),
            pltpu.SemaphoreType.DMA,
            pltpu.SemaphoreType.DMA,
            pltpu.SemaphoreType.DMA,
        ],
    )


def kernel(x, y, F_active, conv_w):
    f_nhwc = jnp.transpose(F_active, (0, 2, 3, 1))
    wt9 = conv_w.reshape(C, 9).T
    table = _conv_tanh(f_nhwc, wt9).reshape(NPIX, C)

    xb = x.reshape(NPTS // 128, 128)
    yb = y.reshape(NPTS // 128, 128)
    idx_out, wts_out = _idxwts(xb, yb)

    idx_pq = idx_out.reshape(4, NPTS).T
    idxs = idx_pq.reshape(NPTS * 4 // 128, 128)
    wtsr = jnp.repeat(wts_out.reshape(8, NPTS).T, 16, axis=1)

    return _sc_interp()(table, idxs, wtsr)


# speedup vs baseline: 1.0464x; 1.0464x over previous
import functools

import numpy as np
import jax
import jax.numpy as jnp
from jax import lax
from jax.experimental import pallas as pl
from jax.experimental.pallas import tpu as pltpu
from jax.experimental.pallas import tpu_sc as plsc

N_CELLS = 2
C = 128
H = 256
W = 256
NPTS = 131072

NC = 2
NS = 16
NW = NC * NS
PTS_PER_W = NPTS // NW
CHUNK = 64
N_CHUNKS = PTS_PER_W // CHUNK
IDX_ROWS = CHUNK * 4 // 128
RPC = CHUNK * 4
NPIX = N_CELLS * H * W

_CONV_T = 8


def _conv_tanh_body(wt_ref, prev_ref, cur_ref, nxt_ref, out_ref):
    i = pl.program_id(1)
    ni = pl.num_programs(1)
    t = _CONV_T
    cur = cur_ref[0]
    top = jnp.where(i > 0, prev_ref[0], 0.0)
    bot = jnp.where(i < ni - 1, nxt_ref[0], 0.0)
    ext = jnp.concatenate([top, cur, bot], axis=0)
    acc = jnp.zeros((t, W, C), jnp.float32)
    zcol = jnp.zeros((t, 1, C), jnp.float32)
    for dh in range(3):
        sl = ext[dh:dh + t]
        for dw in range(3):
            if dw == 0:
                sh = jnp.concatenate([zcol, sl[:, :W - 1, :]], axis=1)
            elif dw == 2:
                sh = jnp.concatenate([sl[:, 1:, :], zcol], axis=1)
            else:
                sh = sl
            acc = acc + sh * wt_ref[dh * 3 + dw][None, None, :]
    u = jax.lax.bitcast_convert_type(jnp.tanh(acc), jnp.uint32)
    bf = jnp.right_shift(
        u + jnp.uint32(0x7FFF)
        + jnp.bitwise_and(jnp.right_shift(u, 16), jnp.uint32(1)), 16)
    g = bf.reshape(t, W // 2, 2, C)
    lo = g[:, :, 0, :]
    hi = g[:, :, 1, :]
    zrow = jnp.zeros((t, 1, C), jnp.uint32)
    lo_next = jnp.concatenate([lo[:, 1:, :], zrow], axis=1)
    e_word = jnp.bitwise_or(lo, jnp.left_shift(hi, 16))
    o_word = jnp.bitwise_or(hi, jnp.left_shift(lo_next, 16))
    out_ref[0, 0] = jax.lax.bitcast_convert_type(e_word, jnp.int32)
    out_ref[1, 0] = jax.lax.bitcast_convert_type(o_word, jnp.int32)


def _conv_tanh(f_nhwc, wt9):
    nblk = H // _CONV_T
    blk = (1, _CONV_T, W, C)
    return pl.pallas_call(
        _conv_tanh_body,
        grid=(N_CELLS, nblk),
        in_specs=[
            pl.BlockSpec((9, C), lambda n, i: (0, 0)),
            pl.BlockSpec((1, 1, W, C),
                         lambda n, i: (n, jnp.maximum(i * _CONV_T - 1, 0), 0, 0)),
            pl.BlockSpec(blk, lambda n, i: (n, i, 0, 0)),
            pl.BlockSpec((1, 1, W, C),
                         lambda n, i: (n, jnp.minimum(i * _CONV_T + _CONV_T,
                                                      H - 1), 0, 0)),
        ],
        out_specs=pl.BlockSpec(
            (2, 1, _CONV_T, W // 2, C), lambda n, i: (0, n, i, 0, 0)),
        out_shape=jax.ShapeDtypeStruct(
            (2, N_CELLS, H, W // 2, C), jnp.int32),
    )(wt9, f_nhwc, f_nhwc, f_nhwc)


def _idxwts_body(x_ref, y_ref, idx_ref, wts_ref):
    xv = x_ref[...]
    yv = y_ref[...]
    ix0 = xv * jnp.float32(W - 2) * 0.5
    iy0 = yv * jnp.float32(H - 2)
    for n in range(N_CELLS):
        off = jnp.float32(n / N_CELLS)
        ix = ix0 + off
        iy = iy0 + off
        ixl = jnp.floor(ix)
        iyt = jnp.floor(iy)
        fx = ix - ixl
        fy = iy - iyt
        wxr = 0.5 - 0.5 * jnp.cos(jnp.pi * fx)
        wxl = 1.0 - wxr
        wyb = 0.5 - 0.5 * jnp.cos(jnp.pi * fy)
        wyt = 1.0 - wyb
        ixl_i = jnp.clip(ixl, 0, W - 1).astype(jnp.int32)
        iyt_i = jnp.clip(iyt, 0, H - 1).astype(jnp.int32)
        p_top = n * H * W + iyt_i * W + ixl_i
        par = jnp.bitwise_and(p_top, 1)
        r_top = jnp.right_shift(p_top, 1) + par * (NPIX // 2)
        idx_ref[2 * n + 0] = r_top
        idx_ref[2 * n + 1] = r_top + W // 2
        wts_ref[4 * n + 0] = wxl * wyt
        wts_ref[4 * n + 1] = wxr * wyt
        wts_ref[4 * n + 2] = wxl * wyb
        wts_ref[4 * n + 3] = wxr * wyb


def _idxwts(xb, yb):
    rows = NPTS // 128
    rblk = 128
    nblk = rows // rblk
    return pl.pallas_call(
        _idxwts_body,
        grid=(nblk,),
        in_specs=[
            pl.BlockSpec((rblk, 128), lambda i: (i, 0)),
            pl.BlockSpec((rblk, 128), lambda i: (i, 0)),
        ],
        out_specs=[
            pl.BlockSpec((4, rblk, 128), lambda i: (0, i, 0)),
            pl.BlockSpec((8, rblk, 128), lambda i: (0, i, 0)),
        ],
        out_shape=[
            jax.ShapeDtypeStruct((4, rows, 128), jnp.int32),
            jax.ShapeDtypeStruct((8, rows, 128), jnp.float32),
        ],
    )(xb, yb)


def _sc_interp_body(table_hbm, idx_hbm, wts_hbm, out_hbm,
                    idx_v, wts_v, rows_v, out_v,
                    sem_idx, sem_gat, sem_out):
    wid = lax.axis_index("s") * NC + lax.axis_index("c")
    iota = lax.iota(jnp.int32, 16)
    n = N_CHUNKS

    def idx_copies(k, b):
        g4 = wid * (N_CHUNKS * IDX_ROWS) + k * IDX_ROWS
        return (
            pltpu.make_async_copy(
                idx_hbm.at[pl.ds(g4, IDX_ROWS)], idx_v.at[b], sem_idx),
            pltpu.make_async_copy(
                wts_hbm.at[pl.ds(wid * PTS_PER_W + k * CHUNK, CHUNK)],
                wts_v.at[pl.ds(b * CHUNK, CHUNK)], sem_idx),
        )

    def gather_copies(b):
        return [pltpu.make_async_copy(
                    table_hbm.at[idx_v.at[b, i]],
                    rows_v.at[pl.ds((b * IDX_ROWS + i) * 128, 128)],
                    sem_gat)
                for i in range(IDX_ROWS)]

    def out_copy(k, b):
        return pltpu.make_async_copy(
            out_v.at[pl.ds(b * CHUNK, CHUNK)],
            out_hbm.at[pl.ds(wid * PTS_PER_W + k * CHUNK, CHUNK)],
            sem_out)

    def compute(b):
        boff = b * CHUNK
        rbase = b * RPC

        @plsc.parallel_loop(0, CHUNK, step=1, unroll=2)
        def pbody(p):
            r0 = rbase + p * 4
            wrow = boff + p
            ws = [wts_v[wrow, pl.ds(j * 16, 16)] for j in range(8)]
            for u in range(8):
                acc = jnp.zeros((16,), jnp.float32)
                for q in range(4):
                    packed = plsc.bitcast(
                        rows_v[r0 + q, pl.ds(u * 16, 16)], jnp.bfloat16)
                    a, bb = plsc.unpack(
                        packed, format=plsc.PackFormat.INTERLEAVED)
                    acc = acc + ws[2 * q] * a + ws[2 * q + 1] * bb
                out_v[wrow, pl.ds(u * 16, 16)] = acc

    for cp in idx_copies(0, 0):
        cp.start()
    for cp in idx_copies(0, 0):
        cp.wait()
    for cp in gather_copies(0):
        cp.start()
    for cp in idx_copies(1, 1):
        cp.start()

    def loop_body(k, carry):
        b = jnp.bitwise_and(k, 1)
        nb = 1 - b
        for cp in gather_copies(b):
            cp.wait()

        @pl.when(k + 1 < n)
        def _():
            for cp in idx_copies(k + 1, nb):
                cp.wait()
            for cp in gather_copies(nb):
                cp.start()

        @pl.when(k >= 2)
        def _():
            out_copy(k - 2, b).wait()

        compute(b)

        @pl.when(k + 2 < n)
        def _():
            for cp in idx_copies(k + 2, b):
                cp.start()

        out_copy(k, b).start()
        return carry

    lax.fori_loop(0, n, loop_body, 0)
    out_copy(n - 2, (n - 2) & 1).wait()
    out_copy(n - 1, (n - 1) & 1).wait()


@functools.lru_cache(maxsize=1)
def _sc_interp():
    return pl.kernel(
        _sc_interp_body,
        out_type=jax.ShapeDtypeStruct((NPTS, C), jnp.float32),
        mesh=plsc.VectorSubcoreMesh(
            core_axis_name="c", subcore_axis_name="s",
            num_cores=NC, num_subcores=NS),
        compiler_params=pltpu.CompilerParams(needs_layout_passes=False),
        scratch_types=[
            pltpu.VMEM((2, IDX_ROWS, 128), jnp.int32),
            pltpu.VMEM((2 * CHUNK, C), jnp.float32),
            pltpu.VMEM((2 * RPC, 128), jnp.int32),
            pltpu.VMEM((2 * CHUNK, C), jnp.float32),
            pltpu.SemaphoreType.DMA,
            pltpu.SemaphoreType.DMA,
            pltpu.SemaphoreType.DMA,
        ],
    )


def kernel(x, y, F_active, conv_w):
    f_nhwc = jnp.transpose(F_active, (0, 2, 3, 1))
    wt9 = conv_w.reshape(C, 9).T
    table = _conv_tanh(f_nhwc, wt9).reshape(NPIX, C)

    xb = x.reshape(NPTS // 128, 128)
    yb = y.reshape(NPTS // 128, 128)
    idx_out, wts_out = _idxwts(xb, yb)

    idx_pq = idx_out.reshape(4, NPTS).T
    idxs = idx_pq.reshape(NPTS * 4 // 128, 128)
    wtsr = jnp.repeat(wts_out.reshape(8, NPTS).T, 16, axis=1)

    return _sc_interp()(table, idxs, wtsr)


# speedup vs baseline: 1.1691x; 1.1173x over previous
import functools

import numpy as np
import jax
import jax.numpy as jnp
from jax import lax
from jax.experimental import pallas as pl
from jax.experimental.pallas import tpu as pltpu
from jax.experimental.pallas import tpu_sc as plsc

N_CELLS = 2
C = 128
H = 256
W = 256
NPTS = 131072

NC = 2
NS = 16
NW = NC * NS
PTS_PER_W = NPTS // NW
CHUNK = 64
N_CHUNKS = PTS_PER_W // CHUNK
IDX_ROWS = CHUNK * 4 // 128
RPC = CHUNK * 4
NPIX = N_CELLS * H * W

_CONV_T = 8


def _conv_tanh_body(wt_ref, prev_ref, cur_ref, nxt_ref, out_ref):
    i = pl.program_id(1)
    ni = pl.num_programs(1)
    t = _CONV_T
    cur = cur_ref[0]
    top = jnp.where(i > 0, prev_ref[0], 0.0)
    bot = jnp.where(i < ni - 1, nxt_ref[0], 0.0)
    ext = jnp.concatenate([top, cur, bot], axis=0)
    acc = jnp.zeros((t, W, C), jnp.float32)
    zcol = jnp.zeros((t, 1, C), jnp.float32)
    for dh in range(3):
        sl = ext[dh:dh + t]
        for dw in range(3):
            if dw == 0:
                sh = jnp.concatenate([zcol, sl[:, :W - 1, :]], axis=1)
            elif dw == 2:
                sh = jnp.concatenate([sl[:, 1:, :], zcol], axis=1)
            else:
                sh = sl
            acc = acc + sh * wt_ref[dh * 3 + dw][None, None, :]
    u = jax.lax.bitcast_convert_type(jnp.tanh(acc), jnp.uint32)
    bf = jnp.right_shift(
        u + jnp.uint32(0x7FFF)
        + jnp.bitwise_and(jnp.right_shift(u, 16), jnp.uint32(1)), 16)
    g = bf.reshape(t, W // 2, 2, C)
    lo = g[:, :, 0, :]
    hi = g[:, :, 1, :]
    zrow = jnp.zeros((t, 1, C), jnp.uint32)
    lo_next = jnp.concatenate([lo[:, 1:, :], zrow], axis=1)
    e_word = jnp.bitwise_or(lo, jnp.left_shift(hi, 16))
    o_word = jnp.bitwise_or(hi, jnp.left_shift(lo_next, 16))
    out_ref[0, 0] = jax.lax.bitcast_convert_type(e_word, jnp.int32)
    out_ref[1, 0] = jax.lax.bitcast_convert_type(o_word, jnp.int32)


def _conv_tanh(f_nhwc, wt9):
    nblk = H // _CONV_T
    blk = (1, _CONV_T, W, C)
    return pl.pallas_call(
        _conv_tanh_body,
        grid=(N_CELLS, nblk),
        in_specs=[
            pl.BlockSpec((9, C), lambda n, i: (0, 0)),
            pl.BlockSpec((1, 1, W, C),
                         lambda n, i: (n, jnp.maximum(i * _CONV_T - 1, 0), 0, 0)),
            pl.BlockSpec(blk, lambda n, i: (n, i, 0, 0)),
            pl.BlockSpec((1, 1, W, C),
                         lambda n, i: (n, jnp.minimum(i * _CONV_T + _CONV_T,
                                                      H - 1), 0, 0)),
        ],
        out_specs=pl.BlockSpec(
            (2, 1, _CONV_T, W // 2, C), lambda n, i: (0, n, i, 0, 0)),
        out_shape=jax.ShapeDtypeStruct(
            (2, N_CELLS, H, W // 2, C), jnp.int32),
    )(wt9, f_nhwc, f_nhwc, f_nhwc)


def _idxwts_body(x_ref, y_ref, idx_ref, wts_ref):
    xv = x_ref[...]
    yv = y_ref[...]
    ix0 = xv * jnp.float32(W - 2) * 0.5
    iy0 = yv * jnp.float32(H - 2)
    for n in range(N_CELLS):
        off = jnp.float32(n / N_CELLS)
        ix = ix0 + off
        iy = iy0 + off
        ixl = jnp.floor(ix)
        iyt = jnp.floor(iy)
        fx = ix - ixl
        fy = iy - iyt
        wxr = 0.5 - 0.5 * jnp.cos(jnp.pi * fx)
        wxl = 1.0 - wxr
        wyb = 0.5 - 0.5 * jnp.cos(jnp.pi * fy)
        wyt = 1.0 - wyb
        ixl_i = jnp.clip(ixl, 0, W - 1).astype(jnp.int32)
        iyt_i = jnp.clip(iyt, 0, H - 1).astype(jnp.int32)
        p_top = n * H * W + iyt_i * W + ixl_i
        par = jnp.bitwise_and(p_top, 1)
        r_top = jnp.right_shift(p_top, 1) + par * (NPIX // 2)
        idx_ref[2 * n + 0] = r_top
        idx_ref[2 * n + 1] = r_top + W // 2
        wts_ref[4 * n + 0] = wxl * wyt
        wts_ref[4 * n + 1] = wxr * wyt
        wts_ref[4 * n + 2] = wxl * wyb
        wts_ref[4 * n + 3] = wxr * wyb


def _idxwts(xb, yb):
    rows = NPTS // 128
    rblk = 128
    nblk = rows // rblk
    return pl.pallas_call(
        _idxwts_body,
        grid=(nblk,),
        in_specs=[
            pl.BlockSpec((rblk, 128), lambda i: (i, 0)),
            pl.BlockSpec((rblk, 128), lambda i: (i, 0)),
        ],
        out_specs=[
            pl.BlockSpec((4, rblk, 128), lambda i: (0, i, 0)),
            pl.BlockSpec((8, rblk, 128), lambda i: (0, i, 0)),
        ],
        out_shape=[
            jax.ShapeDtypeStruct((4, rows, 128), jnp.int32),
            jax.ShapeDtypeStruct((8, rows, 128), jnp.float32),
        ],
    )(xb, yb)


def _sc_interp_body(table_hbm, idx_hbm, wts_hbm, out_hbm,
                    idx_v, wts_v, rows_v, out_v,
                    sem_idx, sem_gat, sem_out):
    wid = lax.axis_index("s") * NC + lax.axis_index("c")
    iota = lax.iota(jnp.int32, 16)
    n = N_CHUNKS

    def wts_copy(k, b):
        return pltpu.make_async_copy(
            wts_hbm.at[pl.ds(wid * PTS_PER_W + k * CHUNK, CHUNK)],
            wts_v.at[pl.ds(b * CHUNK, CHUNK)], sem_idx)

    def gather_copies(k, b):
        return [pltpu.make_async_copy(
                    table_hbm.at[idx_v.at[k * IDX_ROWS + i]],
                    rows_v.at[pl.ds((b * IDX_ROWS + i) * 128, 128)],
                    sem_gat)
                for i in range(IDX_ROWS)]

    def out_copy(k, b):
        return pltpu.make_async_copy(
            out_v.at[pl.ds(b * CHUNK, CHUNK)],
            out_hbm.at[pl.ds(wid * PTS_PER_W + k * CHUNK, CHUNK)],
            sem_out)

    def compute(b):
        boff = b * CHUNK
        rbase = b * RPC

        @plsc.parallel_loop(0, CHUNK, step=1, unroll=2)
        def pbody(p):
            r0 = rbase + p * 4
            wrow = boff + p
            ws = [wts_v[wrow, pl.ds(j * 16, 16)] for j in range(8)]
            for u in range(8):
                acc = jnp.zeros((16,), jnp.float32)
                for q in range(4):
                    packed = plsc.bitcast(
                        rows_v[r0 + q, pl.ds(u * 16, 16)], jnp.bfloat16)
                    a, bb = plsc.unpack(
                        packed, format=plsc.PackFormat.INTERLEAVED)
                    acc = acc + ws[2 * q] * a + ws[2 * q + 1] * bb
                out_v[wrow, pl.ds(u * 16, 16)] = acc

    pltpu.sync_copy(
        idx_hbm.at[pl.ds(wid * (N_CHUNKS * IDX_ROWS), N_CHUNKS * IDX_ROWS)],
        idx_v)
    wts_copy(0, 0).start()
    wts_copy(1, 1).start()
    for cp in gather_copies(0, 0):
        cp.start()

    def loop_body(k, carry):
        b = jnp.bitwise_and(k, 1)
        nb = 1 - b
        for cp in gather_copies(k, b):
            cp.wait()

        @pl.when(k + 1 < n)
        def _():
            for cp in gather_copies(k + 1, nb):
                cp.start()

        @pl.when(k >= 2)
        def _():
            out_copy(k - 2, b).wait()

        wts_copy(k, b).wait()
        compute(b)

        @pl.when(k + 2 < n)
        def _():
            wts_copy(k + 2, b).start()

        out_copy(k, b).start()
        return carry

    lax.fori_loop(0, n, loop_body, 0)
    out_copy(n - 2, (n - 2) & 1).wait()
    out_copy(n - 1, (n - 1) & 1).wait()


@functools.lru_cache(maxsize=1)
def _sc_interp():
    return pl.kernel(
        _sc_interp_body,
        out_type=jax.ShapeDtypeStruct((NPTS, C), jnp.float32),
        mesh=plsc.VectorSubcoreMesh(
            core_axis_name="c", subcore_axis_name="s",
            num_cores=NC, num_subcores=NS),
        compiler_params=pltpu.CompilerParams(needs_layout_passes=False),
        scratch_types=[
            pltpu.VMEM((N_CHUNKS * IDX_ROWS, 128), jnp.int32),
            pltpu.VMEM((2 * CHUNK, C), jnp.float32),
            pltpu.VMEM((2 * RPC, 128), jnp.int32),
            pltpu.VMEM((2 * CHUNK, C), jnp.float32),
            pltpu.SemaphoreType.DMA,
            pltpu.SemaphoreType.DMA,
            pltpu.SemaphoreType.DMA,
        ],
    )


def kernel(x, y, F_active, conv_w):
    f_nhwc = jnp.transpose(F_active, (0, 2, 3, 1))
    wt9 = conv_w.reshape(C, 9).T
    table = _conv_tanh(f_nhwc, wt9).reshape(NPIX, C)

    xb = x.reshape(NPTS // 128, 128)
    yb = y.reshape(NPTS // 128, 128)
    idx_out, wts_out = _idxwts(xb, yb)

    idx_pq = idx_out.reshape(4, NPTS).T
    idxs = idx_pq.reshape(NPTS * 4 // 128, 128)
    wtsr = jnp.repeat(wts_out.reshape(8, NPTS).T, 16, axis=1)

    return _sc_interp()(table, idxs, wtsr)


# speedup vs baseline: 1.1692x; 1.0000x over previous
import functools

import jax
import jax.numpy as jnp
from jax import lax
from jax.experimental import pallas as pl
from jax.experimental.pallas import tpu as pltpu
from jax.experimental.pallas import tpu_sc as plsc

N_CELLS = 2
C = 128
H = 256
W = 256
NPTS = 131072

NC = 2
NS = 16
NW = NC * NS
PTS_PER_W = NPTS // NW
CHUNK = 64
N_CHUNKS = PTS_PER_W // CHUNK
IDX_ROWS = CHUNK * 4 // 128
RPC = CHUNK * 4
NPIX = N_CELLS * H * W

_CONV_T = 8


def _conv_tanh_body(wt_ref, prev_ref, cur_ref, nxt_ref, out_ref):
    i = pl.program_id(1)
    ni = pl.num_programs(1)
    t = _CONV_T
    cur = cur_ref[0]
    top = jnp.where(i > 0, prev_ref[0], 0.0)
    bot = jnp.where(i < ni - 1, nxt_ref[0], 0.0)
    ext = jnp.concatenate([top, cur, bot], axis=0)
    acc = jnp.zeros((t, W, C), jnp.float32)
    zcol = jnp.zeros((t, 1, C), jnp.float32)
    for dh in range(3):
        sl = ext[dh:dh + t]
        for dw in range(3):
            if dw == 0:
                sh = jnp.concatenate([zcol, sl[:, :W - 1, :]], axis=1)
            elif dw == 2:
                sh = jnp.concatenate([sl[:, 1:, :], zcol], axis=1)
            else:
                sh = sl
            acc = acc + sh * wt_ref[dh * 3 + dw][None, None, :]
    u = jax.lax.bitcast_convert_type(jnp.tanh(acc), jnp.uint32)
    bf = jnp.right_shift(
        u + jnp.uint32(0x7FFF)
        + jnp.bitwise_and(jnp.right_shift(u, 16), jnp.uint32(1)), 16)
    g = bf.reshape(t, W // 2, 2, C)
    lo = g[:, :, 0, :]
    hi = g[:, :, 1, :]
    zrow = jnp.zeros((t, 1, C), jnp.uint32)
    lo_next = jnp.concatenate([lo[:, 1:, :], zrow], axis=1)
    e_word = jnp.bitwise_or(lo, jnp.left_shift(hi, 16))
    o_word = jnp.bitwise_or(hi, jnp.left_shift(lo_next, 16))
    out_ref[0, 0] = jax.lax.bitcast_convert_type(e_word, jnp.int32)
    out_ref[1, 0] = jax.lax.bitcast_convert_type(o_word, jnp.int32)


def _conv_tanh(f_nhwc, wt9):
    nblk = H // _CONV_T
    blk = (1, _CONV_T, W, C)
    return pl.pallas_call(
        _conv_tanh_body,
        grid=(N_CELLS, nblk),
        in_specs=[
            pl.BlockSpec((9, C), lambda n, i: (0, 0)),
            pl.BlockSpec((1, 1, W, C),
                         lambda n, i: (n, jnp.maximum(i * _CONV_T - 1, 0), 0, 0)),
            pl.BlockSpec(blk, lambda n, i: (n, i, 0, 0)),
            pl.BlockSpec((1, 1, W, C),
                         lambda n, i: (n, jnp.minimum(i * _CONV_T + _CONV_T,
                                                      H - 1), 0, 0)),
        ],
        out_specs=pl.BlockSpec(
            (2, 1, _CONV_T, W // 2, C), lambda n, i: (0, n, i, 0, 0)),
        out_shape=jax.ShapeDtypeStruct(
            (2, N_CELLS, H, W // 2, C), jnp.int32),
    )(wt9, f_nhwc, f_nhwc, f_nhwc)


def _idxwts_body(x_ref, y_ref, idx_ref, wts_ref):
    xv = x_ref[...]
    yv = y_ref[...]
    ix0 = xv * jnp.float32(W - 2) * 0.5
    iy0 = yv * jnp.float32(H - 2)
    for n in range(N_CELLS):
        off = jnp.float32(n / N_CELLS)
        ix = ix0 + off
        iy = iy0 + off
        ixl = jnp.floor(ix)
        iyt = jnp.floor(iy)
        fx = ix - ixl
        fy = iy - iyt
        wxr = 0.5 - 0.5 * jnp.cos(jnp.pi * fx)
        wxl = 1.0 - wxr
        wyb = 0.5 - 0.5 * jnp.cos(jnp.pi * fy)
        wyt = 1.0 - wyb
        ixl_i = jnp.clip(ixl, 0, W - 1).astype(jnp.int32)
        iyt_i = jnp.clip(iyt, 0, H - 1).astype(jnp.int32)
        p_top = n * H * W + iyt_i * W + ixl_i
        par = jnp.bitwise_and(p_top, 1)
        r_top = jnp.right_shift(p_top, 1) + par * (NPIX // 2)
        idx_ref[2 * n + 0] = r_top
        idx_ref[2 * n + 1] = r_top + W // 2
        wts_ref[4 * n + 0] = wxl * wyt
        wts_ref[4 * n + 1] = wxr * wyt
        wts_ref[4 * n + 2] = wxl * wyb
        wts_ref[4 * n + 3] = wxr * wyb


def _idxwts(xb, yb):
    rows = NPTS // 128
    rblk = 128
    nblk = rows // rblk
    return pl.pallas_call(
        _idxwts_body,
        grid=(nblk,),
        in_specs=[
            pl.BlockSpec((rblk, 128), lambda i: (i, 0)),
            pl.BlockSpec((rblk, 128), lambda i: (i, 0)),
        ],
        out_specs=[
            pl.BlockSpec((4, rblk, 128), lambda i: (0, i, 0)),
            pl.BlockSpec((8, rblk, 128), lambda i: (0, i, 0)),
        ],
        out_shape=[
            jax.ShapeDtypeStruct((4, rows, 128), jnp.int32),
            jax.ShapeDtypeStruct((8, rows, 128), jnp.float32),
        ],
    )(xb, yb)


def _sc_interp_body(table_hbm, idx_hbm, wts_hbm, out_hbm,
                    idx_v, wts_v, rows_v, out_v,
                    sem_idx, sem_gat, sem_out):
    wid = lax.axis_index("s") * NC + lax.axis_index("c")
    n = N_CHUNKS

    def wts_copy(k, b):
        return pltpu.make_async_copy(
            wts_hbm.at[pl.ds(wid * PTS_PER_W + k * CHUNK, CHUNK)],
            wts_v.at[pl.ds(b * CHUNK, CHUNK)], sem_idx)

    def gather_copies(k, b):
        return [pltpu.make_async_copy(
                    table_hbm.at[idx_v.at[k * IDX_ROWS + i]],
                    rows_v.at[pl.ds((b * IDX_ROWS + i) * 128, 128)],
                    sem_gat)
                for i in range(IDX_ROWS)]

    def out_copy(k, b):
        return pltpu.make_async_copy(
            out_v.at[pl.ds(b * CHUNK, CHUNK)],
            out_hbm.at[pl.ds(wid * PTS_PER_W + k * CHUNK, CHUNK)],
            sem_out)

    def compute(b):
        boff = b * CHUNK
        rbase = b * RPC

        @plsc.parallel_loop(0, CHUNK, step=1, unroll=2)
        def pbody(p):
            r0 = rbase + p * 4
            wrow = boff + p
            ws = [wts_v[wrow, pl.ds(j * 16, 16)] for j in range(8)]
            for u in range(8):
                acc = jnp.zeros((16,), jnp.float32)
                for q in range(4):
                    packed = plsc.bitcast(
                        rows_v[r0 + q, pl.ds(u * 16, 16)], jnp.bfloat16)
                    a, bb = plsc.unpack(
                        packed, format=plsc.PackFormat.INTERLEAVED)
                    acc = acc + ws[2 * q] * a + ws[2 * q + 1] * bb
                out_v[wrow, pl.ds(u * 16, 16)] = acc

    pltpu.sync_copy(
        idx_hbm.at[pl.ds(wid * (N_CHUNKS * IDX_ROWS), N_CHUNKS * IDX_ROWS)],
        idx_v)
    wts_copy(0, 0).start()
    wts_copy(1, 1).start()
    for cp in gather_copies(0, 0):
        cp.start()

    def loop_body(k, carry):
        b = jnp.bitwise_and(k, 1)
        nb = 1 - b
        for cp in gather_copies(k, b):
            cp.wait()

        @pl.when(k + 1 < n)
        def _():
            for cp in gather_copies(k + 1, nb):
                cp.start()

        @pl.when(k >= 2)
        def _():
            out_copy(k - 2, b).wait()

        wts_copy(k, b).wait()
        compute(b)

        @pl.when(k + 2 < n)
        def _():
            wts_copy(k + 2, b).start()

        out_copy(k, b).start()
        return carry

    lax.fori_loop(0, n, loop_body, 0)
    out_copy(n - 2, (n - 2) & 1).wait()
    out_copy(n - 1, (n - 1) & 1).wait()


@functools.lru_cache(maxsize=1)
def _sc_interp():
    return pl.kernel(
        _sc_interp_body,
        out_type=jax.ShapeDtypeStruct((NPTS, C), jnp.float32),
        mesh=plsc.VectorSubcoreMesh(
            core_axis_name="c", subcore_axis_name="s",
            num_cores=NC, num_subcores=NS),
        compiler_params=pltpu.CompilerParams(needs_layout_passes=False),
        scratch_types=[
            pltpu.VMEM((N_CHUNKS * IDX_ROWS, 128), jnp.int32),
            pltpu.VMEM((2 * CHUNK, C), jnp.float32),
            pltpu.VMEM((2 * RPC, 128), jnp.int32),
            pltpu.VMEM((2 * CHUNK, C), jnp.float32),
            pltpu.SemaphoreType.DMA,
            pltpu.SemaphoreType.DMA,
            pltpu.SemaphoreType.DMA,
        ],
    )


def kernel(x, y, F_active, conv_w):
    f_nhwc = jnp.transpose(F_active, (0, 2, 3, 1))
    wt9 = conv_w.reshape(C, 9).T
    table = _conv_tanh(f_nhwc, wt9).reshape(NPIX, C)

    xb = x.reshape(NPTS // 128, 128)
    yb = y.reshape(NPTS // 128, 128)
    idx_out, wts_out = _idxwts(xb, yb)

    idx_pq = idx_out.reshape(4, NPTS).T
    idxs = idx_pq.reshape(NPTS * 4 // 128, 128)
    wtsr = jnp.repeat(wts_out.reshape(8, NPTS).T, 16, axis=1)

    return _sc_interp()(table, idxs, wtsr)
